# baseline (speedup 1.0000x reference)
import dataclasses
import functools

import jax
import jax.numpy as jnp
from jax import lax
from jax.experimental import pallas as pl
from jax.experimental.pallas import tpu as pltpu
from jax.experimental.pallas import tpu_sc as plsc

NC = 2
NS = 16
NW = NC * NS
L = 16
K = 64
CHUNK = 128


def _sc_mf_kernel(B):
    b_per_w = B // NW
    n_chunks = b_per_w // CHUNK
    mesh = plsc.VectorSubcoreMesh(core_axis_name="c", subcore_axis_name="s")
    cp = pltpu.CompilerParams()
    if "needs_layout_passes" in pltpu.CompilerParams.__dataclass_fields__:
        cp = dataclasses.replace(cp, needs_layout_passes=False)

    @functools.partial(
        pl.kernel,
        out_type=jax.ShapeDtypeStruct((NW, b_per_w), jnp.float32),
        mesh=mesh,
        compiler_params=cp,
        scratch_types=[
            pltpu.VMEM((n_chunks, CHUNK), jnp.int32),
            pltpu.VMEM((n_chunks, CHUNK), jnp.int32),
            pltpu.VMEM((b_per_w,), jnp.int32),
            pltpu.VMEM((b_per_w,), jnp.int32),
            pltpu.VMEM((CHUNK, 2 * K), jnp.float32),
            pltpu.VMEM((CHUNK, 2 * K), jnp.float32),
            pltpu.VMEM((CHUNK, 2 * K), jnp.float32),
            pltpu.VMEM((CHUNK, 2 * K), jnp.float32),
            pltpu.VMEM((b_per_w,), jnp.float32),
            pltpu.SemaphoreType.DMA,
            pltpu.SemaphoreType.DMA,
        ],
    )
    def kern(u_idx_hbm, i_idx_hbm, u_off_hbm, i_off_hbm, uf_hbm, if_hbm,
             out_hbm, u_idx, i_idx, u_off, i_off, u_a, u_b, v_a, v_b,
             out_v, sem_a, sem_b):
        wid = lax.axis_index("s") * NC + lax.axis_index("c")

        pltpu.sync_copy(u_idx_hbm.at[wid], u_idx)
        pltpu.sync_copy(i_idx_hbm.at[wid], i_idx)
        pltpu.sync_copy(u_off_hbm.at[wid], u_off)
        pltpu.sync_copy(i_off_hbm.at[wid], i_off)

        u_bufs = [u_a, u_b]
        v_bufs = [v_a, v_b]
        sems = [sem_a, sem_b]

        def fire(c):
            s = sems[c % 2]
            return [
                pltpu.async_copy(uf_hbm.at[u_idx.at[c]], u_bufs[c % 2], s),
                pltpu.async_copy(if_hbm.at[i_idx.at[c]], v_bufs[c % 2], s),
            ]

        lane = lax.iota(jnp.int32, L)
        pending = {0: fire(0)}
        for c in range(n_chunks):
            if c + 1 < n_chunks:
                pending[c + 1] = fire(c + 1)
            for cp_ in pending.pop(c):
                cp_.wait()
            u_buf = u_bufs[c % 2]
            v_buf = v_bufs[c % 2]

            @pl.loop(0, CHUNK, step=L)
            def _(r0, c=c, u_buf=u_buf, v_buf=v_buf):
                acc = jnp.zeros((L,), jnp.float32)
                pu_all = u_off[pl.ds(c * CHUNK + r0, L)]
                pv_all = i_off[pl.ds(c * CHUNK + r0, L)]
                for j in range(L):
                    r = r0 + j
                    pu = pu_all[j]
                    pv = pv_all[j]
                    s = (u_buf[r, pl.ds(pu, L)] * v_buf[r, pl.ds(pv, L)]
                         + u_buf[r, pl.ds(pu + L, L)] * v_buf[r, pl.ds(pv + L, L)]
                         + u_buf[r, pl.ds(pu + 2 * L, L)] * v_buf[r, pl.ds(pv + 2 * L, L)]
                         + u_buf[r, pl.ds(pu + 3 * L, L)] * v_buf[r, pl.ds(pv + 3 * L, L)])
                    acc = jnp.where(lane == j, jnp.sum(s), acc)
                out_v[pl.ds(c * CHUNK + r0, L)] = acc

        pltpu.sync_copy(out_v, out_hbm.at[wid])

    return kern


def kernel(user, item, user_factors, item_factors):
    B = user.shape[0]
    b_per_w = B // NW
    n_chunks = b_per_w // CHUNK
    user = user.astype(jnp.int32)
    item = item.astype(jnp.int32)
    uf2 = user_factors.reshape(user_factors.shape[0] // 2, 2 * K)
    if2 = item_factors.reshape(item_factors.shape[0] // 2, 2 * K)
    u_idx = (user >> 1).reshape(NW, n_chunks, CHUNK)
    i_idx = (item >> 1).reshape(NW, n_chunks, CHUNK)
    u_off = ((user & 1) * K).reshape(NW, b_per_w)
    i_off = ((item & 1) * K).reshape(NW, b_per_w)
    out = _sc_mf_kernel(B)(u_idx, i_idx, u_off, i_off, uf2, if2)
    return out.reshape(B)


# speedup vs baseline: 1.6569x; 1.6569x over previous
import dataclasses
import functools

import jax
import jax.numpy as jnp
from jax import lax
from jax.experimental import pallas as pl
from jax.experimental.pallas import tpu as pltpu
from jax.experimental.pallas import tpu_sc as plsc

NC = 2
NS = 16
NW = NC * NS
L = 16
K = 64
CHUNK = 128


def _sc_mf_kernel(B):
    b_per_w = B // NW
    n_chunks = b_per_w // CHUNK
    mesh = plsc.VectorSubcoreMesh(core_axis_name="c", subcore_axis_name="s")
    cp = pltpu.CompilerParams()
    if "needs_layout_passes" in pltpu.CompilerParams.__dataclass_fields__:
        cp = dataclasses.replace(cp, needs_layout_passes=False)

    @functools.partial(
        pl.kernel,
        out_type=jax.ShapeDtypeStruct((NW, b_per_w), jnp.float32),
        mesh=mesh,
        compiler_params=cp,
        scratch_types=[
            pltpu.VMEM((b_per_w,), jnp.int32),
            pltpu.VMEM((b_per_w,), jnp.int32),
            pltpu.VMEM((CHUNK, K), jnp.float32),
            pltpu.VMEM((CHUNK, K), jnp.float32),
            pltpu.VMEM((CHUNK, K), jnp.float32),
            pltpu.VMEM((CHUNK, K), jnp.float32),
            pltpu.VMEM((b_per_w,), jnp.float32),
            pltpu.SemaphoreType.DMA,
            pltpu.SemaphoreType.DMA,
        ],
    )
    def kern(u_idx_hbm, i_idx_hbm, uf_hbm, if_hbm, out_hbm,
             u_idxv, i_idxv, u_a, u_b, v_a, v_b, out_v, sem_a, sem_b):
        wid = lax.axis_index("s") * NC + lax.axis_index("c")

        pltpu.sync_copy(u_idx_hbm.at[wid], u_idxv)
        pltpu.sync_copy(i_idx_hbm.at[wid], i_idxv)

        u_bufs = [u_a, u_b]
        v_bufs = [v_a, v_b]
        sems = [sem_a, sem_b]

        def fire(c):
            u_buf, v_buf, s = u_bufs[c % 2], v_bufs[c % 2], sems[c % 2]

            @pl.loop(0, CHUNK, step=L)
            def _(r0):
                uvec = u_idxv[pl.ds(c * CHUNK + r0, L)]
                ivec = i_idxv[pl.ds(c * CHUNK + r0, L)]
                for j in range(L):
                    pltpu.async_copy(uf_hbm.at[uvec[j]], u_buf.at[r0 + j], s)
                    pltpu.async_copy(if_hbm.at[ivec[j]], v_buf.at[r0 + j], s)

        def drain(c):
            u_buf, v_buf, s = u_bufs[c % 2], v_bufs[c % 2], sems[c % 2]
            pltpu.make_async_copy(uf_hbm.at[pl.ds(0, CHUNK)], u_buf, s).wait()
            pltpu.make_async_copy(if_hbm.at[pl.ds(0, CHUNK)], v_buf, s).wait()

        lane = lax.iota(jnp.int32, L)
        fire(0)
        for c in range(n_chunks):
            if c + 1 < n_chunks:
                fire(c + 1)
            drain(c)
            u_buf = u_bufs[c % 2]
            v_buf = v_bufs[c % 2]

            @pl.loop(0, CHUNK, step=L)
            def _(r0, c=c, u_buf=u_buf, v_buf=v_buf):
                acc = jnp.zeros((L,), jnp.float32)
                for j in range(L):
                    r = r0 + j
                    s = (u_buf[r, pl.ds(0, L)] * v_buf[r, pl.ds(0, L)]
                         + u_buf[r, pl.ds(L, L)] * v_buf[r, pl.ds(L, L)]
                         + u_buf[r, pl.ds(2 * L, L)] * v_buf[r, pl.ds(2 * L, L)]
                         + u_buf[r, pl.ds(3 * L, L)] * v_buf[r, pl.ds(3 * L, L)])
                    acc = jnp.where(lane == j, jnp.sum(s), acc)
                out_v[pl.ds(c * CHUNK + r0, L)] = acc

        pltpu.sync_copy(out_v, out_hbm.at[wid])

    return kern


def kernel(user, item, user_factors, item_factors):
    B = user.shape[0]
    b_per_w = B // NW
    u_idx = user.astype(jnp.int32).reshape(NW, b_per_w)
    i_idx = item.astype(jnp.int32).reshape(NW, b_per_w)
    out = _sc_mf_kernel(B)(u_idx, i_idx, user_factors, item_factors)
    return out.reshape(B)


# speedup vs baseline: 1.6577x; 1.0005x over previous
import dataclasses
import functools

import jax
import jax.numpy as jnp
from jax import lax
from jax.experimental import pallas as pl
from jax.experimental.pallas import tpu as pltpu
from jax.experimental.pallas import tpu_sc as plsc

NC = 2
NS = 16
NW = NC * NS
L = 16
K = 64
CHUNK = 128


def _sc_mf_kernel(B):
    b_per_w = B // NW
    n_chunks = b_per_w // CHUNK
    mesh = plsc.VectorSubcoreMesh(core_axis_name="c", subcore_axis_name="s")
    cp = pltpu.CompilerParams()
    if "needs_layout_passes" in pltpu.CompilerParams.__dataclass_fields__:
        cp = dataclasses.replace(cp, needs_layout_passes=False)
    if "use_tc_tiling_on_sc" in pltpu.CompilerParams.__dataclass_fields__:
        cp = dataclasses.replace(cp, use_tc_tiling_on_sc=True)

    @functools.partial(
        pl.kernel,
        out_type=jax.ShapeDtypeStruct((NW, b_per_w), jnp.float32),
        mesh=mesh,
        compiler_params=cp,
        scratch_types=[
            pltpu.VMEM((b_per_w,), jnp.int32),
            pltpu.VMEM((b_per_w,), jnp.int32),
            pltpu.VMEM((CHUNK, K), jnp.float32),
            pltpu.VMEM((CHUNK, K), jnp.float32),
            pltpu.VMEM((CHUNK, K), jnp.float32),
            pltpu.VMEM((CHUNK, K), jnp.float32),
            pltpu.VMEM((b_per_w,), jnp.float32),
            pltpu.SemaphoreType.DMA,
            pltpu.SemaphoreType.DMA,
        ],
    )
    def kern(u_idx_hbm, i_idx_hbm, uf_hbm, if_hbm, out_hbm,
             u_idxv, i_idxv, u_a, u_b, v_a, v_b, out_v, sem_a, sem_b):
        wid = lax.axis_index("s") * NC + lax.axis_index("c")

        pltpu.sync_copy(u_idx_hbm.at[wid], u_idxv)
        pltpu.sync_copy(i_idx_hbm.at[wid], i_idxv)

        u_bufs = [u_a, u_b]
        v_bufs = [v_a, v_b]
        sems = [sem_a, sem_b]

        def fire(c):
            u_buf, v_buf, s = u_bufs[c % 2], v_bufs[c % 2], sems[c % 2]

            @pl.loop(0, CHUNK, step=L)
            def _(r0):
                uvec = u_idxv[pl.ds(c * CHUNK + r0, L)]
                ivec = i_idxv[pl.ds(c * CHUNK + r0, L)]
                for j in range(L):
                    pltpu.async_copy(uf_hbm.at[uvec[j]], u_buf.at[r0 + j], s)
                    pltpu.async_copy(if_hbm.at[ivec[j]], v_buf.at[r0 + j], s)

        def drain(c):
            u_buf, v_buf, s = u_bufs[c % 2], v_bufs[c % 2], sems[c % 2]
            pltpu.make_async_copy(uf_hbm.at[pl.ds(0, CHUNK)], u_buf, s).wait()
            pltpu.make_async_copy(if_hbm.at[pl.ds(0, CHUNK)], v_buf, s).wait()

        lane = lax.iota(jnp.int32, L)
        fire(0)
        for c in range(n_chunks):
            if c + 1 < n_chunks:
                fire(c + 1)
            drain(c)
            u_buf = u_bufs[c % 2]
            v_buf = v_bufs[c % 2]

            @pl.loop(0, CHUNK, step=L)
            def _(r0, c=c, u_buf=u_buf, v_buf=v_buf):
                acc = jnp.zeros((L,), jnp.float32)
                for j in range(L):
                    r = r0 + j
                    s = (u_buf[r, pl.ds(0, L)] * v_buf[r, pl.ds(0, L)]
                         + u_buf[r, pl.ds(L, L)] * v_buf[r, pl.ds(L, L)]
                         + u_buf[r, pl.ds(2 * L, L)] * v_buf[r, pl.ds(2 * L, L)]
                         + u_buf[r, pl.ds(3 * L, L)] * v_buf[r, pl.ds(3 * L, L)])
                    acc = jnp.where(lane == j, jnp.sum(s), acc)
                out_v[pl.ds(c * CHUNK + r0, L)] = acc

        pltpu.sync_copy(out_v, out_hbm.at[wid])

    return kern


def kernel(user, item, user_factors, item_factors):
    B = user.shape[0]
    b_per_w = B // NW
    u_idx = user.astype(jnp.int32).reshape(NW, b_per_w)
    i_idx = item.astype(jnp.int32).reshape(NW, b_per_w)
    out = _sc_mf_kernel(B)(u_idx, i_idx, user_factors, item_factors)
    return out.reshape(B)
